# Initial kernel scaffold; baseline (speedup 1.0000x reference)
import jax
import jax.numpy as jnp
from jax.experimental import pallas as pl


def kernel(inputs, table):
    raise NotImplementedError("write your pallas kernel here")


# speedup vs baseline: 1.3371x; 1.3371x over previous
import functools

import jax
import jax.numpy as jnp
from jax import lax
from jax.experimental import pallas as pl
from jax.experimental.pallas import tpu as pltpu
from jax.experimental.pallas import tpu_sc as plsc

EMBED = 32
ROW_GRP = 128
NCH = 8
CH = NCH * ROW_GRP
N_WORKERS = 32


def _sc_gather(idx2d, table, n_rows):
    blocks_per_worker = (n_rows // ROW_GRP) // N_WORKERS
    chunks = blocks_per_worker // NCH
    mesh = plsc.VectorSubcoreMesh(core_axis_name="c", subcore_axis_name="s")

    @functools.partial(
        pl.kernel,
        mesh=mesh,
        compiler_params=pltpu.CompilerParams(use_tc_tiling_on_sc=False),
        out_type=jax.ShapeDtypeStruct((n_rows, EMBED), jnp.float32),
        scratch_types=[
            pltpu.VMEM((NCH, ROW_GRP), jnp.int32),
            pltpu.VMEM((CH, EMBED), jnp.float32),
            pltpu.SemaphoreType.DMA,
            pltpu.SemaphoreType.DMA,
        ],
    )
    def k(idx_hbm, table_hbm, out_hbm, idx_v, rows_v, sem_i, sem_g):
        wid = lax.axis_index("s") * 2 + lax.axis_index("c")
        wblk = wid * blocks_per_worker

        def body(c, carry):
            blk = wblk + c * NCH
            pltpu.async_copy(idx_hbm.at[pl.ds(blk, NCH)], idx_v, sem_i).wait()
            cps = [
                pltpu.async_copy(
                    table_hbm.at[idx_v.at[j]],
                    rows_v.at[pl.ds(j * ROW_GRP, ROW_GRP)],
                    sem_g,
                )
                for j in range(NCH)
            ]
            for cp in cps:
                cp.wait()
            pltpu.sync_copy(rows_v, out_hbm.at[pl.ds(blk * ROW_GRP, CH)])
            return carry

        lax.fori_loop(0, chunks, body, 0)

    return k(idx2d, table)


def kernel(inputs, table):
    b, l = inputs.shape
    n = b * l
    idx2d = inputs.T.reshape(n // ROW_GRP, ROW_GRP)
    out = _sc_gather(idx2d, table, n)
    return out.reshape(l, b, EMBED)


# speedup vs baseline: 1.3732x; 1.0270x over previous
import functools

import jax
import jax.numpy as jnp
from jax import lax
from jax.experimental import pallas as pl
from jax.experimental.pallas import tpu as pltpu
from jax.experimental.pallas import tpu_sc as plsc

EMBED = 32
ROW_GRP = 128
NCH = 10
CH = NCH * ROW_GRP
N_WORKERS = 32


def _sc_gather(idx2d, table, n_rows):
    blocks_per_worker = (n_rows // ROW_GRP) // N_WORKERS
    chunks = blocks_per_worker // NCH
    mesh = plsc.VectorSubcoreMesh(core_axis_name="c", subcore_axis_name="s")

    @functools.partial(
        pl.kernel,
        mesh=mesh,
        compiler_params=pltpu.CompilerParams(use_tc_tiling_on_sc=False),
        out_type=jax.ShapeDtypeStruct((n_rows, EMBED), jnp.float32),
        scratch_types=[
            pltpu.VMEM((blocks_per_worker, ROW_GRP), jnp.int32),
            pltpu.VMEM((CH, EMBED), jnp.float32),
            pltpu.VMEM((CH, EMBED), jnp.float32),
            pltpu.SemaphoreType.DMA,
            pltpu.SemaphoreType.DMA,
            pltpu.SemaphoreType.DMA,
            pltpu.SemaphoreType.DMA,
            pltpu.SemaphoreType.DMA,
        ],
    )
    def k(idx_hbm, table_hbm, out_hbm, idx_v, rows0, rows1,
          sem_i, sg0, sg1, sw0, sw1):
        wid = lax.axis_index("s") * 2 + lax.axis_index("c")
        wblk = wid * blocks_per_worker
        wrow = wblk * ROW_GRP

        pltpu.async_copy(
            idx_hbm.at[pl.ds(wblk, blocks_per_worker)], idx_v, sem_i
        ).wait()

        def fire(c, rows, sg):
            for j in range(NCH):
                pltpu.async_copy(
                    table_hbm.at[idx_v.at[c * NCH + j]],
                    rows.at[pl.ds(j * ROW_GRP, ROW_GRP)],
                    sg,
                )

        def wait_gathers(rows, sg):
            for j in range(NCH):
                pltpu.make_async_copy(
                    table_hbm.at[idx_v.at[j]],
                    rows.at[pl.ds(j * ROW_GRP, ROW_GRP)],
                    sg,
                ).wait()

        def start_write(c, rows, sw):
            pltpu.async_copy(rows, out_hbm.at[pl.ds(wrow + c * CH, CH)], sw)

        def wait_write(rows, sw):
            pltpu.make_async_copy(rows, out_hbm.at[pl.ds(wrow, CH)], sw).wait()

        fire(0, rows0, sg0)

        def body2(i, carry):
            c0 = 2 * i
            c1 = c0 + 1

            @pl.when(i > 0)
            def _():
                wait_write(rows1, sw1)

            fire(c1, rows1, sg1)
            wait_gathers(rows0, sg0)
            start_write(c0, rows0, sw0)

            @pl.when(c1 + 1 < chunks)
            def _():
                wait_write(rows0, sw0)
                fire(c1 + 1, rows0, sg0)

            wait_gathers(rows1, sg1)
            start_write(c1, rows1, sw1)
            return carry

        lax.fori_loop(0, chunks // 2, body2, 0)
        wait_write(rows1, sw1)

    return k(idx2d, table)


def kernel(inputs, table):
    b, l = inputs.shape
    n = b * l
    idx2d = inputs.T.reshape(n // ROW_GRP, ROW_GRP)
    out = _sc_gather(idx2d, table, n)
    return out.reshape(l, b, EMBED)
